# Initial kernel scaffold; baseline (speedup 1.0000x reference)
import functools

import jax
import jax.numpy as jnp
from jax import lax
from jax.experimental import pallas as pl
from jax.experimental.pallas import tpu as pltpu
from jax.experimental.pallas import tpu_sc as plsc

_B, _L, _E = 4096, 50, 64
_N = _B * _L
_NC, _NS = 2, 16
_NW = _NC * _NS
_PER_W = _N // _NW
_CHUNK = 800
_NCHUNK = _PER_W // _CHUNK


def _make_kernel():
  mesh = plsc.VectorSubcoreMesh(
      core_axis_name="c", subcore_axis_name="s",
      num_cores=_NC, num_subcores=_NS)

  @functools.partial(
      pl.kernel,
      out_type=(
          jax.ShapeDtypeStruct((_N, _E), jnp.float32),
          jax.ShapeDtypeStruct((_N, _E), jnp.float32),
      ),
      mesh=mesh,
      scratch_types=[
          pltpu.VMEM((2, _CHUNK), jnp.int32),
          pltpu.VMEM((2, _CHUNK, _E), jnp.float32),
          pltpu.SemaphoreType.DMA,
          pltpu.SemaphoreType.DMA,
      ],
  )
  def emb_kernel(src_idx, tar_idx, src_table, tar_table,
                 src_out, tar_out, idx_v, rows_v, sem0, sem1):
    wid = lax.axis_index("s") * _NC + lax.axis_index("c")
    base = wid * _PER_W

    def body(i, _):
      off = base + i * _CHUNK
      pltpu.sync_copy(src_idx.at[pl.ds(off, _CHUNK)], idx_v.at[0])
      pltpu.sync_copy(tar_idx.at[pl.ds(off, _CHUNK)], idx_v.at[1])
      g0 = pltpu.async_copy(src_table.at[idx_v.at[0]], rows_v.at[0], sem0)
      g1 = pltpu.async_copy(tar_table.at[idx_v.at[1]], rows_v.at[1], sem1)
      g0.wait()
      g1.wait()
      pltpu.sync_copy(rows_v.at[0], src_out.at[pl.ds(off, _CHUNK)])
      pltpu.sync_copy(rows_v.at[1], tar_out.at[pl.ds(off, _CHUNK)])
      return 0

    lax.fori_loop(0, _NCHUNK, body, 0)

  return emb_kernel


_EMB = _make_kernel()


@jax.jit
def kernel(src_idx, tar_idx, src_table, tar_table):
  src_flat = src_idx.reshape(_N)
  tar_flat = tar_idx.reshape(_N)
  src_out, tar_out = _EMB(src_flat, tar_flat, src_table, tar_table)
  return (src_out.reshape(_B, _L, _E), tar_out.reshape(_B, _L, _E))


# speedup vs baseline: 4.8820x; 4.8820x over previous
import functools

import jax
import jax.numpy as jnp
from jax import lax
from jax.experimental import pallas as pl
from jax.experimental.pallas import tpu as pltpu
from jax.experimental.pallas import tpu_sc as plsc

_B, _L, _E = 4096, 50, 64
_N = _B * _L
_NC, _NS = 2, 16
_NW = _NC * _NS
_PER_W = _N // _NW
_CHUNK = 800
_NCHUNK = _PER_W // _CHUNK


def _make_kernel():
  mesh = plsc.VectorSubcoreMesh(
      core_axis_name="c", subcore_axis_name="s",
      num_cores=_NC, num_subcores=_NS)

  @functools.partial(
      pl.kernel,
      out_type=(
          jax.ShapeDtypeStruct((_N, _E), jnp.float32),
          jax.ShapeDtypeStruct((_N, _E), jnp.float32),
      ),
      mesh=mesh,
      compiler_params=pltpu.CompilerParams(use_tc_tiling_on_sc=False),
      scratch_types=[
          pltpu.VMEM((_CHUNK,), jnp.int32),
          pltpu.VMEM((_CHUNK,), jnp.int32),
          pltpu.VMEM((_CHUNK, _E), jnp.float32),
          pltpu.VMEM((_CHUNK, _E), jnp.float32),
          pltpu.SemaphoreType.DMA,
          pltpu.SemaphoreType.DMA,
      ],
  )
  def emb_kernel(src_idx, tar_idx, src_table, tar_table,
                 src_out, tar_out, idx0, idx1, rows0, rows1, sem0, sem1):
    wid = lax.axis_index("s") * _NC + lax.axis_index("c")
    base = wid * _PER_W

    def body(i, _):
      off = base + i * _CHUNK
      pltpu.sync_copy(src_idx.at[pl.ds(off, _CHUNK)], idx0)
      pltpu.sync_copy(tar_idx.at[pl.ds(off, _CHUNK)], idx1)
      g0 = pltpu.async_copy(src_table.at[idx0], rows0, sem0)
      g1 = pltpu.async_copy(tar_table.at[idx1], rows1, sem1)
      g0.wait()
      g1.wait()
      pltpu.sync_copy(rows0, src_out.at[pl.ds(off, _CHUNK)])
      pltpu.sync_copy(rows1, tar_out.at[pl.ds(off, _CHUNK)])
      return 0

    lax.fori_loop(0, _NCHUNK, body, 0)

  return emb_kernel


_EMB = _make_kernel()


@jax.jit
def kernel(src_idx, tar_idx, src_table, tar_table):
  src_flat = src_idx.reshape(_N)
  tar_flat = tar_idx.reshape(_N)
  src_out, tar_out = _EMB(src_flat, tar_flat, src_table, tar_table)
  return (src_out.reshape(_B, _L, _E), tar_out.reshape(_B, _L, _E))


# speedup vs baseline: 5.0024x; 1.0247x over previous
import functools

import jax
import jax.numpy as jnp
from jax import lax
from jax.experimental import pallas as pl
from jax.experimental.pallas import tpu as pltpu
from jax.experimental.pallas import tpu_sc as plsc

_B, _L, _E = 4096, 50, 64
_N = _B * _L
_NC, _NS = 2, 16
_NW = _NC * _NS
_PER_W = _N // _NW
_CHUNK = 400
_NCHUNK = _PER_W // _CHUNK


def _make_kernel():
  mesh = plsc.VectorSubcoreMesh(
      core_axis_name="c", subcore_axis_name="s",
      num_cores=_NC, num_subcores=_NS)

  @functools.partial(
      pl.kernel,
      out_type=(
          jax.ShapeDtypeStruct((_N, _E), jnp.float32),
          jax.ShapeDtypeStruct((_N, _E), jnp.float32),
      ),
      mesh=mesh,
      compiler_params=pltpu.CompilerParams(use_tc_tiling_on_sc=False),
      scratch_types=[
          pltpu.VMEM((_PER_W,), jnp.int32),
          pltpu.VMEM((_PER_W,), jnp.int32),
          pltpu.VMEM((_CHUNK, _E), jnp.float32),
          pltpu.VMEM((_CHUNK, _E), jnp.float32),
          pltpu.VMEM((_CHUNK, _E), jnp.float32),
          pltpu.VMEM((_CHUNK, _E), jnp.float32),
          pltpu.SemaphoreType.DMA,
          pltpu.SemaphoreType.DMA,
          pltpu.SemaphoreType.DMA,
          pltpu.SemaphoreType.DMA,
      ],
  )
  def emb_kernel(src_idx, tar_idx, src_table, tar_table,
                 src_out, tar_out, idxs, idxt,
                 rs0, rt0, rs1, rt1, ss0, st0, ss1, st1):
    wid = lax.axis_index("s") * _NC + lax.axis_index("c")
    base = wid * _PER_W
    pltpu.sync_copy(src_idx.at[pl.ds(base, _PER_W)], idxs)
    pltpu.sync_copy(tar_idx.at[pl.ds(base, _PER_W)], idxt)

    bufs = ((rs0, rt0, ss0, st0), (rs1, rt1, ss1, st1))

    def start(c, buf):
      rs, rt, ss, st = bufs[buf]
      o = c * _CHUNK
      pltpu.async_copy(src_table.at[idxs.at[pl.ds(o, _CHUNK)]], rs, ss)
      pltpu.async_copy(tar_table.at[idxt.at[pl.ds(o, _CHUNK)]], rt, st)

    def finish(c, buf):
      rs, rt, ss, st = bufs[buf]
      o = base + c * _CHUNK
      pltpu.make_async_copy(src_table.at[pl.ds(0, _CHUNK)], rs, ss).wait()
      pltpu.make_async_copy(tar_table.at[pl.ds(0, _CHUNK)], rt, st).wait()
      pltpu.sync_copy(rs, src_out.at[pl.ds(o, _CHUNK)])
      pltpu.sync_copy(rt, tar_out.at[pl.ds(o, _CHUNK)])

    start(0, 0)

    def body(i, _):
      c = 2 * i
      start(c + 1, 1)
      finish(c, 0)
      start(c + 2, 0)
      finish(c + 1, 1)
      return 0

    lax.fori_loop(0, _NCHUNK // 2 - 1, body, 0)

    start(_NCHUNK - 1, 1)
    finish(_NCHUNK - 2, 0)
    finish(_NCHUNK - 1, 1)

  return emb_kernel


_EMB = _make_kernel()


@jax.jit
def kernel(src_idx, tar_idx, src_table, tar_table):
  src_flat = src_idx.reshape(_N)
  tar_flat = tar_idx.reshape(_N)
  src_out, tar_out = _EMB(src_flat, tar_flat, src_table, tar_table)
  return (src_out.reshape(_B, _L, _E), tar_out.reshape(_B, _L, _E))
